# baseline (speedup 1.0000x reference)
import functools

import jax
import jax.numpy as jnp
from jax import lax
from jax.experimental import pallas as pl
from jax.experimental.pallas import tpu as pltpu
from jax.experimental.pallas import tpu_sc as plsc

VOCAB = 100000
EMBED_DIM = 64
BATCH = 4096
SEQ = 26
SEQ_PAD = 32
NUM_WORKERS = 32
SLABS_PER_WORKER = BATCH // NUM_WORKERS
GROUP = 8
NBUF = 4

_MESH = plsc.VectorSubcoreMesh(core_axis_name="c", subcore_axis_name="s")


@functools.partial(
    pl.kernel,
    out_type=jax.ShapeDtypeStruct((BATCH, SEQ, EMBED_DIM), jnp.float32),
    mesh=_MESH,
    scratch_types=[
        pltpu.VMEM((SLABS_PER_WORKER * SEQ_PAD,), jnp.int32),
        pltpu.VMEM((SLABS_PER_WORKER * SEQ_PAD,), jnp.int32),
        pltpu.VMEM((NBUF, SEQ_PAD, 2 * EMBED_DIM), jnp.float32),
        pltpu.VMEM((2, GROUP, SEQ, EMBED_DIM), jnp.float32),
        pltpu.SemaphoreType.DMA((NBUF,)),
        pltpu.SemaphoreType.DMA((2,)),
    ],
)
def _gather_kernel(table_hbm, idx_hbm, out_hbm, idx_v, pair_v, pbufs, obufs,
                   gsem, wsem):
    wid = lax.axis_index("s") * 2 + lax.axis_index("c")
    sbase = wid * SLABS_PER_WORKER

    n_idx = SLABS_PER_WORKER * SEQ_PAD
    pltpu.sync_copy(idx_hbm.at[pl.ds(wid * n_idx, n_idx)], idx_v)

    def halve(k, _):
        v = idx_v[pl.ds(k * 16, 16)]
        pair_v[pl.ds(k * 16, 16)] = lax.shift_right_logical(v, 1)
        return _
    lax.fori_loop(0, n_idx // 16, halve, None)

    def fire(slab, pb):
        return pltpu.async_copy(
            table_hbm.at[pair_v.at[pl.ds(slab * SEQ_PAD, SEQ_PAD)]],
            pbufs.at[pb], gsem.at[pb])

    for b in range(NBUF):
        fire(b, b)

    def body(slab, _):
        pb = lax.rem(slab, NBUF)
        grp = slab // GROUP
        s = lax.rem(slab, GROUP)
        ob = lax.rem(grp, 2)

        @pl.when(jnp.logical_and(s == 0, grp >= 2))
        def _():
            pltpu.make_async_copy(out_hbm.at[pl.ds(0, GROUP)], obufs.at[ob],
                                  wsem.at[ob]).wait()

        pltpu.make_async_copy(table_hbm.at[pl.ds(0, SEQ_PAD)], pbufs.at[pb],
                              gsem.at[pb]).wait()

        par = idx_v[pl.ds(slab * SEQ_PAD, 16)] & 1
        par_hi = idx_v[pl.ds(slab * SEQ_PAD + 16, 16)] & 1
        for half, count in ((0, 16), (1, SEQ - 16)):
            pvec = par if half == 0 else par_hi
            for r in range(count):
                rr = half * 16 + r
                m = jnp.take(pvec, jnp.full((16,), r, jnp.int32))
                mf = m.astype(jnp.float32)
                for q in range(EMBED_DIM // 16):
                    lo = pbufs[pb, rr, pl.ds(q * 16, 16)]
                    hi = pbufs[pb, rr, pl.ds(EMBED_DIM + q * 16, 16)]
                    obufs[ob, s, rr, pl.ds(q * 16, 16)] = (
                        lo + (hi - lo) * mf)

        @pl.when(slab + NBUF < SLABS_PER_WORKER)
        def _():
            fire(slab + NBUF, pb)

        @pl.when(s == GROUP - 1)
        def _():
            pltpu.async_copy(
                obufs.at[ob],
                out_hbm.at[pl.ds(sbase + grp * GROUP, GROUP)], wsem.at[ob])
        return _

    lax.fori_loop(0, SLABS_PER_WORKER, body, None)

    for ob in range(2):
        pltpu.make_async_copy(out_hbm.at[pl.ds(0, GROUP)], obufs.at[ob],
                              wsem.at[ob]).wait()


def kernel(indices, table):
    idx = jnp.pad(indices.astype(jnp.int32), ((0, 0), (0, SEQ_PAD - SEQ)))
    tbl2 = table.reshape(VOCAB // 2, 2 * EMBED_DIM)
    return _gather_kernel(tbl2, idx.reshape(-1))


# speedup vs baseline: 1.0016x; 1.0016x over previous
import functools

import jax
import jax.numpy as jnp
from jax import lax
from jax.experimental import pallas as pl
from jax.experimental.pallas import tpu as pltpu
from jax.experimental.pallas import tpu_sc as plsc

VOCAB = 100000
EMBED_DIM = 64
BATCH = 4096
SEQ = 26
SEQ_PAD = 32
NUM_WORKERS = 32
SLABS_PER_WORKER = BATCH // NUM_WORKERS
GROUP = 8
NBUF = 4

_MESH = plsc.VectorSubcoreMesh(core_axis_name="c", subcore_axis_name="s")


@functools.partial(
    pl.kernel,
    out_type=jax.ShapeDtypeStruct((BATCH, SEQ, EMBED_DIM), jnp.float32),
    mesh=_MESH,
    scratch_types=[
        pltpu.VMEM((SLABS_PER_WORKER * SEQ_PAD,), jnp.int32),
        pltpu.VMEM((SLABS_PER_WORKER * SEQ_PAD,), jnp.int32),
        pltpu.VMEM((NBUF, SEQ_PAD, 2 * EMBED_DIM), jnp.float32),
        pltpu.VMEM((2, GROUP, SEQ, EMBED_DIM), jnp.float32),
        pltpu.SemaphoreType.DMA((NBUF,)),
        pltpu.SemaphoreType.DMA((2,)),
    ],
)
def _gather_kernel(table_hbm, idx_hbm, out_hbm, idx_v, pair_v, pbufs, obufs,
                   gsem, wsem):
    wid = lax.axis_index("s") * 2 + lax.axis_index("c")
    sbase = wid * SLABS_PER_WORKER

    n_idx = SLABS_PER_WORKER * SEQ_PAD
    pltpu.sync_copy(idx_hbm.at[pl.ds(wid * n_idx, n_idx)], idx_v)

    def halve(k, _):
        v = idx_v[pl.ds(k * 16, 16)]
        pair_v[pl.ds(k * 16, 16)] = lax.shift_right_logical(v, 1)
        return _
    lax.fori_loop(0, n_idx // 16, halve, None)

    def fire(slab, pb):
        return pltpu.async_copy(
            table_hbm.at[pair_v.at[pl.ds(slab * SEQ_PAD, SEQ_PAD)]],
            pbufs.at[pb], gsem.at[pb])

    for b in range(NBUF):
        fire(b, b)

    def body(gg, _):
        for og in range(2):
            grp = 2 * gg + og

            @pl.when(gg > 0)
            def _():
                pltpu.make_async_copy(out_hbm.at[pl.ds(0, GROUP)],
                                      obufs.at[og], wsem.at[og]).wait()

            for s in range(GROUP):
                slab = grp * GROUP + s
                pb = (og * GROUP + s) % NBUF

                pltpu.make_async_copy(table_hbm.at[pl.ds(0, SEQ_PAD)],
                                      pbufs.at[pb], gsem.at[pb]).wait()

                for half, count in ((0, 16), (1, SEQ - 16)):
                    pvec = idx_v[pl.ds(slab * SEQ_PAD + half * 16, 16)] & 1
                    for r in range(count):
                        rr = half * 16 + r
                        m = jnp.take(pvec, jnp.full((16,), r, jnp.int32))
                        mf = m.astype(jnp.float32)
                        for q in range(EMBED_DIM // 16):
                            lo = pbufs[pb, rr, pl.ds(q * 16, 16)]
                            hi = pbufs[pb, rr,
                                       pl.ds(EMBED_DIM + q * 16, 16)]
                            obufs[og, s, rr, pl.ds(q * 16, 16)] = (
                                lo + (hi - lo) * mf)

                @pl.when(slab + NBUF < SLABS_PER_WORKER)
                def _():
                    fire(slab + NBUF, pb)

            pltpu.async_copy(
                obufs.at[og],
                out_hbm.at[pl.ds(sbase + grp * GROUP, GROUP)], wsem.at[og])
        return _

    lax.fori_loop(0, SLABS_PER_WORKER // (2 * GROUP), body, None)

    for ob in range(2):
        pltpu.make_async_copy(out_hbm.at[pl.ds(0, GROUP)], obufs.at[ob],
                              wsem.at[ob]).wait()


def kernel(indices, table):
    idx = jnp.pad(indices.astype(jnp.int32), ((0, 0), (0, SEQ_PAD - SEQ)))
    tbl2 = table.reshape(VOCAB // 2, 2 * EMBED_DIM)
    return _gather_kernel(tbl2, idx.reshape(-1))


# speedup vs baseline: 6.7884x; 6.7777x over previous
import functools

import jax
import jax.numpy as jnp
from jax import lax
from jax.experimental import pallas as pl
from jax.experimental.pallas import tpu as pltpu
from jax.experimental.pallas import tpu_sc as plsc

VOCAB = 100000
EMBED_DIM = 64
BATCH = 4096
SEQ = 26
NUM_WORKERS = 32
SLABS_PER_WORKER = BATCH // NUM_WORKERS
GROUP = 16
GROUPS_PER_WORKER = SLABS_PER_WORKER // GROUP

_MESH = plsc.VectorSubcoreMesh(core_axis_name="c", subcore_axis_name="s")


@functools.partial(
    pl.kernel,
    out_type=jax.ShapeDtypeStruct((BATCH, SEQ, EMBED_DIM), jnp.float32),
    mesh=_MESH,
    compiler_params=pltpu.CompilerParams(use_tc_tiling_on_sc=False),
    scratch_types=[
        pltpu.VMEM((SLABS_PER_WORKER, SEQ), jnp.int32),
        pltpu.VMEM((GROUP, SEQ, EMBED_DIM), jnp.float32),
        pltpu.VMEM((GROUP, SEQ, EMBED_DIM), jnp.float32),
        pltpu.SemaphoreType.DMA,
        pltpu.SemaphoreType.DMA,
        pltpu.SemaphoreType.DMA,
        pltpu.SemaphoreType.DMA,
    ],
)
def _gather_kernel(table_hbm, idx_hbm, out_hbm, idx_v, buf0, buf1,
                   g0, g1, w0, w1):
    wid = lax.axis_index("s") * 2 + lax.axis_index("c")
    sbase = wid * SLABS_PER_WORKER

    pltpu.sync_copy(idx_hbm.at[pl.ds(sbase, SLABS_PER_WORKER)], idx_v)

    bufs = (buf0, buf1)
    gsems = (g0, g1)
    wsems = (w0, w1)

    def body(i, _):
        gcopies = [[None] * GROUP, [None] * GROUP]
        for p in range(2):
            g = 2 * i + p

            @pl.when(i > 0)
            def _():
                pltpu.make_async_copy(out_hbm.at[pl.ds(0, GROUP)], bufs[p],
                                      wsems[p]).wait()

            for s in range(GROUP):
                gcopies[p][s] = pltpu.async_copy(
                    table_hbm.at[idx_v.at[g * GROUP + s]],
                    bufs[p].at[s], gsems[p])
        for p in range(2):
            g = 2 * i + p
            for s in range(GROUP):
                gcopies[p][s].wait()
            pltpu.async_copy(bufs[p],
                             out_hbm.at[pl.ds(sbase + g * GROUP, GROUP)],
                             wsems[p])
        return _

    lax.fori_loop(0, GROUPS_PER_WORKER // 2, body, None)

    for p in range(2):
        pltpu.make_async_copy(out_hbm.at[pl.ds(0, GROUP)], bufs[p],
                              wsems[p]).wait()


def kernel(indices, table):
    idx = indices.astype(jnp.int32)
    return _gather_kernel(table, idx)
